# Initial kernel scaffold; baseline (speedup 1.0000x reference)
import jax
import jax.numpy as jnp
from jax.experimental import pallas as pl


def kernel(agg_msg, node_type, W_att, b_att):
    raise NotImplementedError("write your pallas kernel here")


# speedup vs baseline: 10.0812x; 10.0812x over previous
import jax
import jax.numpy as jnp
from jax.experimental import pallas as pl

N = 10000
T = 8
IN = 128
OUT = 128
B = 1000


def _agg_kernel(oh_ref, x_ref, w_ref, b_ref, o_ref):
    x = jnp.maximum(x_ref[...], 0.0)
    oh = oh_ref[...]
    acc = jnp.dot(oh, b_ref[...], preferred_element_type=jnp.float32)
    for t in range(T):
        y = jnp.dot(x, w_ref[t], preferred_element_type=jnp.float32)
        acc = acc + y * oh[:, t:t + 1]
    o_ref[...] = jnp.maximum(acc, 0.0)


def kernel(agg_msg, node_type, W_att, b_att):
    x = agg_msg.reshape(N, IN)
    oh = jax.nn.one_hot(node_type, T, dtype=jnp.float32)
    grid = (N // B,)
    out = pl.pallas_call(
        _agg_kernel,
        grid=grid,
        in_specs=[
            pl.BlockSpec((B, T), lambda i: (i, 0)),
            pl.BlockSpec((B, IN), lambda i: (i, 0)),
            pl.BlockSpec((T, IN, OUT), lambda i: (0, 0, 0)),
            pl.BlockSpec((T, OUT), lambda i: (0, 0)),
        ],
        out_specs=pl.BlockSpec((B, OUT), lambda i: (i, 0)),
        out_shape=jax.ShapeDtypeStruct((N, OUT), jnp.float32),
    )(oh, x, W_att, b_att)
    return out
